# baseline (speedup 1.0000x reference)
import functools

import jax
import jax.numpy as jnp
from jax import lax
from jax.experimental import pallas as pl
from jax.experimental.pallas import tpu as pltpu
from jax.experimental.pallas import tpu_sc as plsc

BBLK = 128
NROW = 4
NSLAB = 3
GDEPTH = 3


@functools.partial(jax.jit, static_argnames=("length", "emb", "ntc"))
def _sc_gather(idx2, table, *, length, emb, ntc):
    mesh = plsc.VectorSubcoreMesh(core_axis_name="c", subcore_axis_name="s")
    nc = mesh.num_cores
    nw = nc * mesh.num_subcores
    ntasks = length * ntc
    per_w = ntasks // nw
    eh = emb // 8
    n_m = (emb * BBLK) // 16
    bm = BBLK // 16

    def body(table_hbm, idx_hbm, out_hbm, idx_v, rows_v, slab_v, gsem, wsem):
        wid = lax.axis_index("s") * nc + lax.axis_index("c")
        t0 = wid * per_w
        pltpu.sync_copy(idx_hbm.at[pl.ds(t0, per_w)], idx_v)

        def gather_desc(j, buf):
            return pltpu.make_async_copy(
                table_hbm.at[idx_v.at[j]], rows_v.at[buf], gsem
            )

        def write_desc(j, sb):
            t = t0 + j
            return pltpu.make_async_copy(
                slab_v.at[sb],
                out_hbm.at[lax.div(t, ntc), :, lax.rem(t, ntc)],
                wsem,
            )

        lane = lax.iota(jnp.int32, 16)

        def permute(buf, sb):
            @pl.loop(0, n_m, unroll=8)
            def _(m):
                e = lax.div(m, bm)
                b0 = lax.rem(m, bm) * 16
                v = plsc.load_gather(
                    rows_v.at[buf], [b0 + lane, jnp.full((16,), e, jnp.int32)]
                )
                slab_v[sb, lax.div(e, 8), lax.rem(e, 8), pl.ds(b0, 16)] = v

        for p in range(GDEPTH):
            gather_desc(p, p).start()

        @pl.loop(0, per_w)
        def _(j):
            gather_desc(j, lax.rem(j, NROW)).wait()

            @pl.when(j >= NSLAB)
            def _():
                write_desc(j - NSLAB, lax.rem(j - NSLAB, NSLAB)).wait()

            sb = lax.rem(j, NSLAB)
            permute(lax.rem(j, NROW), sb)
            write_desc(j, sb).start()

            @pl.when(j + GDEPTH < per_w)
            def _():
                jn = j + GDEPTH
                gather_desc(jn, lax.rem(jn, NROW)).start()

        @pl.loop(0, NSLAB)
        def _(t):
            jj = per_w - NSLAB + t
            write_desc(jj, lax.rem(jj, NSLAB)).wait()

    run = pl.kernel(
        body,
        out_type=jax.ShapeDtypeStruct((length, eh, ntc, 8, BBLK), jnp.float32),
        mesh=mesh,
        compiler_params=pltpu.CompilerParams(
            use_tc_tiling_on_sc=False, needs_layout_passes=False
        ),
        scratch_types=[
            pltpu.VMEM((per_w, BBLK), jnp.int32),
            pltpu.VMEM((NROW, BBLK, emb), jnp.float32),
            pltpu.VMEM((NSLAB, eh, 8, BBLK), jnp.float32),
            pltpu.SemaphoreType.DMA,
            pltpu.SemaphoreType.DMA,
        ],
    )
    return run(table, idx2)


def kernel(inputs, shared_weights):
    bsz, length = inputs.shape
    vocab, emb = shared_weights.shape
    ntc = bsz // BBLK
    assert ntc * BBLK == bsz and emb % 8 == 0
    idx = inputs if inputs.dtype == jnp.int32 else inputs.astype(jnp.int32)
    idx2 = idx.T.reshape(length * ntc, BBLK)
    out5 = _sc_gather(idx2, shared_weights, length=length, emb=emb, ntc=ntc)
    return out5.transpose(2, 4, 0, 1, 3).reshape(bsz, length, emb)
